# baseline (speedup 1.0000x reference)
import jax
import jax.numpy as jnp
from jax import lax
from jax.experimental import pallas as pl
from jax.experimental.pallas import tpu as pltpu
from jax.experimental.pallas import tpu_sc as plsc

N_ROWS = 4096
N_COLS = 1000
STRIDE = 5
N_OUT_COLS = N_COLS // STRIDE

_TOTAL_IN = N_ROWS * N_COLS
_TOTAL_OUT = N_ROWS * N_OUT_COLS
_NC = 2
_NS = 16
_NW = _NC * _NS

_IN_PER_W = _TOTAL_IN // _NW
_OUT_PER_W = _TOTAL_OUT // _NW
_NCHUNK = 4
_CIN = _IN_PER_W // _NCHUNK
_COUT = _OUT_PER_W // _NCHUNK
_VECS = _COUT // 16


def _body(x_hbm, out_hbm, in_v, out_v):
    wid = lax.axis_index("s") * _NC + lax.axis_index("c")
    base_in = wid * _IN_PER_W
    base_out = wid * _OUT_PER_W
    lane_off = lax.iota(jnp.int32, 16) * STRIDE

    def chunk(c, carry):
        pltpu.sync_copy(x_hbm.at[pl.ds(base_in + c * _CIN, _CIN)], in_v)

        def vec(v, carry2):
            g = plsc.load_gather(in_v, [lane_off + v * (16 * STRIDE)])
            out_v[pl.ds(v * 16, 16)] = g
            return carry2

        lax.fori_loop(0, _VECS, vec, 0, unroll=8)
        pltpu.sync_copy(out_v, out_hbm.at[pl.ds(base_out + c * _COUT, _COUT)])
        return carry

    lax.fori_loop(0, _NCHUNK, chunk, 0)


_sc_gather = pl.kernel(
    _body,
    mesh=plsc.VectorSubcoreMesh(core_axis_name="c", subcore_axis_name="s"),
    out_type=jax.ShapeDtypeStruct((_TOTAL_OUT,), jnp.float32),
    scratch_types=[
        pltpu.VMEM((_CIN,), jnp.float32),
        pltpu.VMEM((_COUT,), jnp.float32),
    ],
    compiler_params=pltpu.CompilerParams(needs_layout_passes=False),
)


@jax.jit
def kernel(x, mask):
    del mask
    flat = _sc_gather(x.reshape(_TOTAL_IN))
    return flat.reshape(N_ROWS, N_OUT_COLS)
